# baseline (speedup 1.0000x reference)
import functools

import jax
import jax.numpy as jnp
from jax import lax
from jax.experimental import pallas as pl
from jax.experimental.pallas import tpu as pltpu
from jax.experimental.pallas import tpu_sc as plsc

NUM_CORES = 2
NUM_SUBCORES = 16
LANES = 16
NUM_WORKERS = NUM_CORES * NUM_SUBCORES

BATCH = 16384
PER_WORKER = BATCH // NUM_WORKERS
TABLE = 1001

_mesh = plsc.VectorSubcoreMesh(core_axis_name="c", subcore_axis_name="s")


@functools.partial(
    pl.kernel,
    mesh=_mesh,
    out_type=jax.ShapeDtypeStruct((BATCH,), jnp.float32),
    scratch_types=[
        pltpu.VMEM((TABLE,), jnp.float32),
        pltpu.VMEM((PER_WORKER,), jnp.int32),
        pltpu.VMEM((PER_WORKER,), jnp.float32),
        pltpu.SemaphoreType.DMA,
    ],
    compiler_params=pltpu.CompilerParams(needs_layout_passes=False),
)
def _gather_sc(betas_hbm, idx_hbm, out_hbm, tbl_v, idx_v, out_v, sem):
    wid = lax.axis_index("s") * NUM_CORES + lax.axis_index("c")
    base = wid * PER_WORKER
    pltpu.async_copy(betas_hbm, tbl_v, sem)
    cp_idx = pltpu.async_copy(idx_hbm.at[pl.ds(base, PER_WORKER)], idx_v, sem)
    cp_idx.wait()
    pltpu.make_async_copy(betas_hbm, tbl_v, sem).wait()
    @pl.loop(0, PER_WORKER // LANES)
    def _gather_loop(i):
        idx = idx_v[pl.ds(i * LANES, LANES)]
        out_v[pl.ds(i * LANES, LANES)] = plsc.load_gather(tbl_v, [idx])
    pltpu.sync_copy(out_v, out_hbm.at[pl.ds(base, PER_WORKER)])


def kernel(betas, t_int):
    return _gather_sc(betas.astype(jnp.float32), t_int.astype(jnp.int32))


# speedup vs baseline: 1.0040x; 1.0040x over previous
import functools

import jax
import jax.numpy as jnp
from jax import lax
from jax.experimental import pallas as pl
from jax.experimental.pallas import tpu as pltpu
from jax.experimental.pallas import tpu_sc as plsc

NUM_CORES = 2
NUM_SUBCORES = 16
LANES = 16
NUM_WORKERS = NUM_CORES * NUM_SUBCORES

BATCH = 16384
PER_WORKER = BATCH // NUM_WORKERS
TABLE = 1001

_mesh = plsc.VectorSubcoreMesh(core_axis_name="c", subcore_axis_name="s")


@functools.partial(
    pl.kernel,
    mesh=_mesh,
    out_type=jax.ShapeDtypeStruct((BATCH,), jnp.float32),
    scratch_types=[
        pltpu.VMEM((TABLE,), jnp.float32),
        pltpu.VMEM((PER_WORKER,), jnp.int32),
        pltpu.VMEM((PER_WORKER,), jnp.float32),
        pltpu.SemaphoreType.DMA,
        pltpu.SemaphoreType.DMA,
    ],
    compiler_params=pltpu.CompilerParams(needs_layout_passes=False),
)
def _gather_sc(betas_hbm, idx_hbm, out_hbm, tbl_v, idx_v, out_v, sem0, sem1):
    wid = lax.axis_index("s") * NUM_CORES + lax.axis_index("c")
    base = wid * PER_WORKER
    cp_tbl = pltpu.async_copy(betas_hbm, tbl_v, sem0)
    cp_idx = pltpu.async_copy(idx_hbm.at[pl.ds(base, PER_WORKER)], idx_v, sem1)
    cp_tbl.wait()
    cp_idx.wait()
    @pl.loop(0, PER_WORKER // LANES)
    def _gather_loop(i):
        idx = idx_v[pl.ds(i * LANES, LANES)]
        out_v[pl.ds(i * LANES, LANES)] = plsc.load_gather(tbl_v, [idx])
    pltpu.sync_copy(out_v, out_hbm.at[pl.ds(base, PER_WORKER)])


def kernel(betas, t_int):
    return _gather_sc(betas.astype(jnp.float32), t_int.astype(jnp.int32))
